# baseline (speedup 1.0000x reference)
import functools

import jax
import jax.numpy as jnp
from jax import lax
from jax.experimental import pallas as pl
from jax.experimental.pallas import tpu as pltpu
from jax.experimental.pallas import tpu_sc as plsc

VOCAB = 1000000
DIM = 32
B = 4096
L = 50
N = B * L
NC = 2
NS = 16
NW = NC * NS
PER_W = N // NW
CH = 128
STEPS = PER_W // CH
ROW = DIM * 4

_mesh = plsc.VectorSubcoreMesh(core_axis_name="c", subcore_axis_name="s")


@functools.partial(
    pl.kernel,
    out_type=jax.ShapeDtypeStruct((N, ROW), jnp.float32),
    mesh=_mesh,
    scratch_types=[
        pltpu.VMEM((CH,), jnp.int32),
        pltpu.VMEM((CH, DIM), jnp.float32),
        pltpu.VMEM((CH, DIM), jnp.float32),
        pltpu.VMEM((CH, DIM), jnp.float32),
        pltpu.VMEM((CH, DIM), jnp.float32),
        pltpu.VMEM((CH, ROW), jnp.float32),
        pltpu.SemaphoreType.DMA,
    ],
    compiler_params=pltpu.CompilerParams(
        needs_layout_passes=False, use_tc_tiling_on_sc=False),
)
def _emb(x_hbm, s_hbm, vi_hbm, vj_hbm, vk_hbm, out_hbm,
         idx_v, r0, r1, r2, r3, obuf, sem):
    wid = lax.axis_index("s") * NC + lax.axis_index("c")
    lanes = lax.iota(jnp.int32, 16)
    rows = (r0, r1, r2, r3)
    tables = (s_hbm, vi_hbm, vj_hbm, vk_hbm)

    def step(t, carry):
        base = wid * PER_W + t * CH
        pltpu.sync_copy(x_hbm.at[pl.ds(base, CH)], idx_v)
        copies = [pltpu.async_copy(tb.at[idx_v], rb, sem)
                  for tb, rb in zip(tables, rows)]
        for c in copies:
            c.wait()

        def row(r, carry2):
            r_idx = jnp.full((16,), r, jnp.int32)
            for q in range(4):
                for h in range(2):
                    v = rows[q][r, pl.ds(16 * h, 16)]
                    plsc.store_scatter(
                        obuf, [r_idx, (64 * h + q) + 4 * lanes], v)
            return carry2

        lax.fori_loop(0, CH, row, 0, unroll=2)
        pltpu.sync_copy(obuf, out_hbm.at[pl.ds(base, CH)])
        return carry

    lax.fori_loop(0, STEPS, step, 0)


def kernel(x, scalar, vector_i, vector_j, vector_k):
    xf = x.reshape(-1).astype(jnp.int32)
    out = _emb(xf, scalar, vector_i, vector_j, vector_k)
    return out.reshape(B, L, DIM, 4)


# speedup vs baseline: 1.0499x; 1.0499x over previous
import functools

import jax
import jax.numpy as jnp
from jax import lax
from jax.experimental import pallas as pl
from jax.experimental.pallas import tpu as pltpu
from jax.experimental.pallas import tpu_sc as plsc

VOCAB = 1000000
DIM = 32
B = 4096
L = 50
N = B * L
NC = 2
NS = 16
NW = NC * NS
PER_W = N // NW
CH = 128
STEPS = PER_W // CH
ROW = DIM * 4

_mesh = plsc.VectorSubcoreMesh(core_axis_name="c", subcore_axis_name="s")


@functools.partial(
    pl.kernel,
    out_type=jax.ShapeDtypeStruct((N, ROW), jnp.float32),
    mesh=_mesh,
    scratch_types=[
        pltpu.VMEM((STEPS, CH), jnp.int32),
        [pltpu.VMEM((4, CH, DIM), jnp.float32) for _ in range(2)],
        [pltpu.VMEM((CH, ROW), jnp.float32) for _ in range(2)],
        [pltpu.SemaphoreType.DMA for _ in range(2)],
        [pltpu.SemaphoreType.DMA for _ in range(2)],
    ],
    compiler_params=pltpu.CompilerParams(
        needs_layout_passes=False, use_tc_tiling_on_sc=False),
)
def _emb(x_hbm, s_hbm, vi_hbm, vj_hbm, vk_hbm, out_hbm,
         idx_all, rbufs, obufs, gsems, osems):
    wid = lax.axis_index("s") * NC + lax.axis_index("c")
    lanes = lax.iota(jnp.int32, 16)
    tables = (s_hbm, vi_hbm, vj_hbm, vk_hbm)
    cols = [(64 * h + q) + 4 * lanes for q in range(4) for h in range(2)]

    pltpu.sync_copy(x_hbm.at[pl.ds(wid * STEPS, STEPS)], idx_all)

    def fire(t, bset):
        for q in range(4):
            pltpu.make_async_copy(
                tables[q].at[idx_all.at[t]], rbufs[bset].at[q],
                gsems[bset]).start()

    def gwait(bset):
        for q in range(4):
            pltpu.make_async_copy(
                tables[q].at[idx_all.at[0]], rbufs[bset].at[q],
                gsems[bset]).wait()

    fire(0, 0)

    def pair(k, carry):
        for bset in range(2):
            t = 2 * k + bset

            @pl.when(t + 1 < STEPS)
            def _():
                fire(t + 1, 1 - bset)

            gwait(bset)

            @pl.when(k > 0)
            def _():
                pltpu.make_async_copy(
                    obufs[bset], out_hbm.at[pl.ds(0, CH)],
                    osems[bset]).wait()

            rbuf, obuf = rbufs[bset], obufs[bset]

            def row(r, carry2):
                r_idx = jnp.full((16,), r, jnp.int32)
                for q in range(4):
                    for h in range(2):
                        v = rbuf[q, r, pl.ds(16 * h, 16)]
                        plsc.store_scatter(
                            obuf, [r_idx, cols[2 * q + h]], v)
                return carry2

            lax.fori_loop(0, CH, row, 0, unroll=4)

            base = wid * PER_W + t * CH
            pltpu.make_async_copy(
                obuf, out_hbm.at[pl.ds(base, CH)], osems[bset]).start()
        return carry

    lax.fori_loop(0, STEPS // 2, pair, 0)

    for bset in range(2):
        pltpu.make_async_copy(
            obufs[bset], out_hbm.at[pl.ds(0, CH)], osems[bset]).wait()


def kernel(x, scalar, vector_i, vector_j, vector_k):
    x2d = x.reshape(N // CH, CH).astype(jnp.int32)
    out = _emb(x2d, scalar, vector_i, vector_j, vector_k)
    return out.reshape(B, L, DIM, 4)


# speedup vs baseline: 1.2069x; 1.1496x over previous
import functools

import jax
import jax.numpy as jnp
from jax import lax
from jax.experimental import pallas as pl
from jax.experimental.pallas import tpu as pltpu
from jax.experimental.pallas import tpu_sc as plsc

VOCAB = 1000000
DIM = 32
B = 4096
L = 50
N = B * L
NC = 2
NS = 16
NW = NC * NS
PER_W = N // NW
CH = 64
STEPS = PER_W // CH
ROW = DIM * 4

_mesh = plsc.VectorSubcoreMesh(core_axis_name="c", subcore_axis_name="s")


@functools.partial(
    pl.kernel,
    out_type=jax.ShapeDtypeStruct((N, ROW), jnp.float32),
    mesh=_mesh,
    scratch_types=[
        [pltpu.VMEM((CH,), jnp.int32) for _ in range(2)],
        [[pltpu.VMEM((CH, DIM), jnp.float32) for _ in range(4)]
         for _ in range(2)],
        pltpu.VMEM((CH, ROW), jnp.float32),
        [pltpu.SemaphoreType.DMA for _ in range(2)],
        pltpu.SemaphoreType.DMA,
    ],
    compiler_params=pltpu.CompilerParams(needs_layout_passes=False),
)
def _emb(x_hbm, s_hbm, vi_hbm, vj_hbm, vk_hbm, out_hbm,
         idxbs, rbufs, obuf, gsems, osem):
    wid = lax.axis_index("s") * NC + lax.axis_index("c")
    lanes = lax.iota(jnp.int32, 16)
    tables = (s_hbm, vi_hbm, vj_hbm, vk_hbm)
    cols = [(64 * h + q) + 4 * lanes for q in range(4) for h in range(2)]

    def fire(t, bset):
        pltpu.sync_copy(x_hbm.at[pl.ds(wid * PER_W + t * CH, CH)],
                        idxbs[bset])

        def grp(g, carry):
            vec = idxbs[bset][pl.ds(g * 16, 16)]
            for l in range(16):
                v = vec[l]
                for q in range(4):
                    pltpu.make_async_copy(
                        tables[q].at[pl.ds(v, 1)],
                        rbufs[bset][q].at[pl.ds(g * 16 + l, 1)],
                        gsems[bset]).start()
            return carry

        lax.fori_loop(0, CH // 16, grp, 0)

    def gwait(bset):
        for q in range(4):
            pltpu.make_async_copy(
                tables[q].at[pl.ds(0, CH)], rbufs[bset][q],
                gsems[bset]).wait()

    fire(0, 0)

    def pair(k, carry):
        for bset in range(2):
            t = 2 * k + bset

            @pl.when(t + 1 < STEPS)
            def _():
                fire(t + 1, 1 - bset)

            gwait(bset)

            @pl.when(t > 0)
            def _():
                pltpu.make_async_copy(
                    obuf, out_hbm.at[pl.ds(0, CH)], osem).wait()

            rbuf = rbufs[bset]

            def row(r, carry2):
                r_idx = jnp.full((16,), r, jnp.int32)
                for q in range(4):
                    for h in range(2):
                        v = rbuf[q][r, pl.ds(16 * h, 16)]
                        plsc.store_scatter(
                            obuf, [r_idx, cols[2 * q + h]], v)
                return carry2

            lax.fori_loop(0, CH, row, 0, unroll=4)

            base = wid * PER_W + t * CH
            pltpu.make_async_copy(
                obuf, out_hbm.at[pl.ds(base, CH)], osem).start()
        return carry

    lax.fori_loop(0, STEPS // 2, pair, 0)

    pltpu.make_async_copy(
        obuf, out_hbm.at[pl.ds(0, CH)], osem).wait()


def kernel(x, scalar, vector_i, vector_j, vector_k):
    xf = x.reshape(-1).astype(jnp.int32)
    out = _emb(xf, scalar, vector_i, vector_j, vector_k)
    return out.reshape(B, L, DIM, 4)
